# Initial kernel scaffold; baseline (speedup 1.0000x reference)
import jax
import jax.numpy as jnp
from jax.experimental import pallas as pl


def kernel(token_ids, emb_table):
    raise NotImplementedError("write your pallas kernel here")


# speedup vs baseline: 3.3209x; 3.3209x over previous
import functools

import jax
import jax.numpy as jnp
from jax import lax
from jax.experimental import pallas as pl
from jax.experimental.pallas import tpu as pltpu
from jax.experimental.pallas import tpu_sc as plsc

D = 128
NC, NS = 2, 16
NW = NC * NS
CHUNK = 128


@functools.lru_cache(maxsize=None)
def _emb_call(b_total: int, vocab: int):
    assert b_total % (NW * CHUNK) == 0
    b_per_w = b_total // NW
    n_chunks = b_per_w // CHUNK

    mesh = plsc.VectorSubcoreMesh(
        core_axis_name="c", subcore_axis_name="s",
        num_cores=NC, num_subcores=NS,
    )

    @functools.partial(
        pl.kernel,
        out_type=jax.ShapeDtypeStruct((b_total, D), jnp.float32),
        mesh=mesh,
        scratch_types=[
            pltpu.VMEM((n_chunks, CHUNK), jnp.int32),
            pltpu.VMEM((CHUNK, D), jnp.float32),
            pltpu.VMEM((CHUNK, D), jnp.float32),
            pltpu.SemaphoreType.DMA,
            pltpu.SemaphoreType.DMA,
        ],
    )
    def emb(idx_hbm, table_hbm, out_hbm, idx_v, rows0, rows1, g0, g1):
        wid = lax.axis_index("s") * NC + lax.axis_index("c")
        base = wid * b_per_w

        pltpu.sync_copy(idx_hbm.at[wid], idx_v)

        bufs = (rows0, rows1)
        sems = (g0, g1)

        pltpu.async_copy(table_hbm.at[idx_v.at[0]], bufs[0], sems[0])
        pltpu.async_copy(table_hbm.at[idx_v.at[1]], bufs[1], sems[1])

        @pl.loop(0, n_chunks, step=2)
        def _(j):
            for b in range(2):
                c = j + b
                pltpu.make_async_copy(
                    table_hbm.at[idx_v.at[c]], bufs[b], sems[b]
                ).wait()
                pltpu.sync_copy(
                    bufs[b], out_hbm.at[pl.ds(base + c * CHUNK, CHUNK)]
                )

                @pl.when(c + 2 < n_chunks)
                def _():
                    pltpu.async_copy(
                        table_hbm.at[idx_v.at[c + 2]], bufs[b], sems[b]
                    )

    return emb


def kernel(token_ids, emb_table):
    b, l = token_ids.shape
    vocab, d = emb_table.shape
    assert d == D
    ids = token_ids.astype(jnp.int32).reshape(NW, -1, CHUNK)
    out = _emb_call(b * l, vocab)(ids, emb_table)
    return out.reshape(b, l, D)


# speedup vs baseline: 3.3384x; 1.0053x over previous
import functools

import jax
import jax.numpy as jnp
from jax import lax
from jax.experimental import pallas as pl
from jax.experimental.pallas import tpu as pltpu
from jax.experimental.pallas import tpu_sc as plsc

D = 128
NC, NS = 2, 16
NW = NC * NS
CHUNK = 128


@functools.lru_cache(maxsize=None)
def _emb_call(b_total: int, vocab: int):
    assert b_total % (NW * CHUNK) == 0
    b_per_w = b_total // NW
    n_chunks = b_per_w // CHUNK

    mesh = plsc.VectorSubcoreMesh(
        core_axis_name="c", subcore_axis_name="s",
        num_cores=NC, num_subcores=NS,
    )

    NBUF = 5
    LOOK = 3
    assert n_chunks % NBUF == 0

    @functools.partial(
        pl.kernel,
        out_type=jax.ShapeDtypeStruct((b_total, D), jnp.float32),
        mesh=mesh,
        scratch_types=[
            pltpu.VMEM((n_chunks, CHUNK), jnp.int32),
            [pltpu.VMEM((CHUNK, D), jnp.float32) for _ in range(NBUF)],
            [pltpu.SemaphoreType.DMA for _ in range(NBUF)],
            [pltpu.SemaphoreType.DMA for _ in range(NBUF)],
        ],
    )
    def emb(idx_hbm, table_hbm, out_hbm, idx_v, bufs, gsem, ssem):
        wid = lax.axis_index("s") * NC + lax.axis_index("c")
        base = wid * b_per_w

        pltpu.sync_copy(idx_hbm.at[wid], idx_v)

        def out_slice(c):
            return out_hbm.at[pl.ds(base + c * CHUNK, CHUNK)]

        for b in range(LOOK):
            pltpu.async_copy(table_hbm.at[idx_v.at[b]], bufs[b], gsem[b])

        @pl.loop(0, n_chunks, step=NBUF)
        def _(j):
            for b in range(NBUF):
                c = j + b
                pltpu.make_async_copy(
                    table_hbm.at[idx_v.at[c]], bufs[b], gsem[b]
                ).wait()
                pltpu.async_copy(bufs[b], out_slice(c), ssem[b])

                bb = (b + LOOK) % NBUF

                @pl.when(c + LOOK < n_chunks)
                def _():
                    @pl.when(c >= NBUF - LOOK)
                    def _():
                        pltpu.make_async_copy(
                            bufs[bb], out_slice(c), ssem[bb]
                        ).wait()

                    pltpu.async_copy(
                        table_hbm.at[idx_v.at[c + LOOK]], bufs[bb], gsem[bb]
                    )

        for b in range(NBUF):
            pltpu.make_async_copy(bufs[b], out_slice(b), ssem[b]).wait()

    return emb


def kernel(token_ids, emb_table):
    b, l = token_ids.shape
    vocab, d = emb_table.shape
    assert d == D
    ids = token_ids.astype(jnp.int32).reshape(NW, -1, CHUNK)
    out = _emb_call(b * l, vocab)(ids, emb_table)
    return out.reshape(b, l, D)


# speedup vs baseline: 5.9351x; 1.7778x over previous
import functools

import jax
import jax.numpy as jnp
from jax import lax
from jax.experimental import pallas as pl
from jax.experimental.pallas import tpu as pltpu
from jax.experimental.pallas import tpu_sc as plsc

D = 128
NC, NS = 2, 16
NW = NC * NS
NB = 2


@functools.lru_cache(maxsize=None)
def _emb_call(batch: int, seq: int, vocab: int):
    assert batch % (NW * NB) == 0
    rows_per_chunk = NB * seq
    assert rows_per_chunk <= 128
    batches_per_w = batch // NW
    n_chunks = batches_per_w // NB

    NBUF = 4
    LOOK = 3
    assert n_chunks % NBUF == 0

    mesh = plsc.VectorSubcoreMesh(
        core_axis_name="c", subcore_axis_name="s",
        num_cores=NC, num_subcores=NS,
    )

    @functools.partial(
        pl.kernel,
        out_type=jax.ShapeDtypeStruct((batch, seq, D), jnp.float32),
        mesh=mesh,
        scratch_types=[
            pltpu.VMEM((n_chunks, rows_per_chunk), jnp.int32),
            [pltpu.VMEM((rows_per_chunk, D), jnp.float32) for _ in range(NBUF)],
            [pltpu.SemaphoreType.DMA for _ in range(NBUF)],
            [pltpu.SemaphoreType.DMA for _ in range(NBUF)],
        ],
    )
    def emb(idx_hbm, table_hbm, out_hbm, idx_v, bufs, gsem, ssem):
        wid = lax.axis_index("s") * NC + lax.axis_index("c")
        base = wid * batches_per_w

        pltpu.sync_copy(idx_hbm.at[wid], idx_v)

        def fire_gather(c, b):
            pltpu.async_copy(table_hbm.at[idx_v.at[c]], bufs[b], gsem[b])

        def fire_stores(c, b):
            for k in range(NB):
                pltpu.async_copy(
                    bufs[b].at[pl.ds(k * seq, seq)],
                    out_hbm.at[base + c * NB + k],
                    ssem[b],
                )

        def drain_stores(c, b):
            for k in range(NB):
                pltpu.make_async_copy(
                    bufs[b].at[pl.ds(k * seq, seq)],
                    out_hbm.at[base + c * NB + k],
                    ssem[b],
                ).wait()

        for b in range(LOOK):
            fire_gather(b, b)

        @pl.loop(0, n_chunks, step=NBUF)
        def _(j):
            for b in range(NBUF):
                c = j + b
                pltpu.make_async_copy(
                    table_hbm.at[idx_v.at[c]], bufs[b], gsem[b]
                ).wait()
                fire_stores(c, b)

                bb = (b + LOOK) % NBUF

                @pl.when(c + LOOK < n_chunks)
                def _():
                    @pl.when(c >= NBUF - LOOK)
                    def _():
                        drain_stores(c, bb)

                    fire_gather(c + LOOK, bb)

        for b in range(NBUF):
            drain_stores(0, b)

    return emb


def kernel(token_ids, emb_table):
    batch, seq = token_ids.shape
    vocab, d = emb_table.shape
    assert d == D
    ids = token_ids.astype(jnp.int32).reshape(NW, -1, NB * seq)
    return _emb_call(batch, seq, vocab)(ids, emb_table)


# speedup vs baseline: 5.9470x; 1.0020x over previous
import functools

import jax
import jax.numpy as jnp
from jax import lax
from jax.experimental import pallas as pl
from jax.experimental.pallas import tpu as pltpu
from jax.experimental.pallas import tpu_sc as plsc

D = 128
NC, NS = 2, 16
NW = NC * NS
NB = 2


@functools.lru_cache(maxsize=None)
def _emb_call(batch: int, seq: int, vocab: int):
    assert batch % (NW * NB) == 0
    rows_per_chunk = NB * seq
    assert rows_per_chunk <= 128
    batches_per_w = batch // NW
    n_chunks = batches_per_w // NB

    NBUF = 4
    LOOK = 3
    assert n_chunks % NBUF == 0

    mesh = plsc.VectorSubcoreMesh(
        core_axis_name="c", subcore_axis_name="s",
        num_cores=NC, num_subcores=NS,
    )

    @functools.partial(
        pl.kernel,
        out_type=jax.ShapeDtypeStruct((batch, seq, D), jnp.float32),
        mesh=mesh,
        compiler_params=pltpu.CompilerParams(use_tc_tiling_on_sc=True),
        scratch_types=[
            pltpu.VMEM((n_chunks, 128), jnp.int32),
            [pltpu.VMEM((rows_per_chunk, D), jnp.float32) for _ in range(NBUF)],
            [pltpu.SemaphoreType.DMA for _ in range(NBUF)],
            [pltpu.SemaphoreType.DMA for _ in range(NBUF)],
        ],
    )
    def emb(idx_hbm, table_hbm, out_hbm, idx_v, bufs, gsem, ssem):
        wid = lax.axis_index("s") * NC + lax.axis_index("c")
        base = wid * batches_per_w

        pltpu.sync_copy(idx_hbm.at[wid], idx_v)

        def fire_gather(c, b):
            pltpu.async_copy(
                table_hbm.at[idx_v.at[c, pl.ds(0, rows_per_chunk)]],
                bufs[b], gsem[b],
            )

        def fire_stores(c, b):
            for k in range(NB):
                pltpu.async_copy(
                    bufs[b].at[pl.ds(k * seq, seq)],
                    out_hbm.at[base + c * NB + k],
                    ssem[b],
                )

        def drain_stores(c, b):
            for k in range(NB):
                pltpu.make_async_copy(
                    bufs[b].at[pl.ds(k * seq, seq)],
                    out_hbm.at[base + c * NB + k],
                    ssem[b],
                ).wait()

        for b in range(LOOK):
            fire_gather(b, b)

        @pl.loop(0, n_chunks, step=NBUF)
        def _(j):
            for b in range(NBUF):
                c = j + b
                pltpu.make_async_copy(
                    table_hbm.at[idx_v.at[c, pl.ds(0, rows_per_chunk)]],
                    bufs[b], gsem[b],
                ).wait()
                fire_stores(c, b)

                bb = (b + LOOK) % NBUF

                @pl.when(c + LOOK < n_chunks)
                def _():
                    @pl.when(c >= NBUF - LOOK)
                    def _():
                        drain_stores(c, bb)

                    fire_gather(c + LOOK, bb)

        for b in range(NBUF):
            drain_stores(0, b)

    return emb


def kernel(token_ids, emb_table):
    batch, seq = token_ids.shape
    vocab, d = emb_table.shape
    assert d == D
    ids = token_ids.astype(jnp.int32).reshape(NW, -1, NB * seq)
    ids = jnp.pad(ids, ((0, 0), (0, 0), (0, 128 - NB * seq)))
    return _emb_call(batch, seq, vocab)(ids, emb_table)
